# speedup vs baseline: 4.8843x; 4.8843x over previous
import jax
import jax.numpy as jnp
from jax.experimental import pallas as pl
from jax.experimental.pallas import tpu as pltpu

_HIDDEN = 30
_LANE = 128
_BLOCK_B = 4096


def _anet_fused_kernel(x_ref, w1_ref, b1_ref, w2_ref, b2_ref, o_ref):
    h = jnp.dot(x_ref[...], w1_ref[...], preferred_element_type=jnp.float32)
    h = jnp.maximum(h + b1_ref[...], 0.0)
    y = jnp.dot(h, w2_ref[...], preferred_element_type=jnp.float32)
    o_ref[...] = jnp.tanh(y + b2_ref[...]) * 2.0


def kernel(x, w1, b1, w2, b2):
    B, s_dim = x.shape
    a_dim = w2.shape[1]
    x = x.astype(jnp.float32)

    h_pad = max(_LANE, ((_HIDDEN + _LANE - 1) // _LANE) * _LANE)
    w1p = jnp.zeros((s_dim, h_pad), jnp.float32).at[:, :_HIDDEN].set(
        w1.astype(jnp.float32))
    b1p = jnp.zeros((1, h_pad), jnp.float32).at[:, :_HIDDEN].set(
        jnp.reshape(b1, (1, -1)).astype(jnp.float32))
    w2p = jnp.zeros((h_pad, a_dim), jnp.float32).at[:_HIDDEN, :].set(
        w2.astype(jnp.float32))
    b2p = jnp.reshape(b2, (1, a_dim)).astype(jnp.float32)

    block_b = min(_BLOCK_B, B)
    pad_b = (-B) % block_b
    if pad_b:
        x = jnp.pad(x, ((0, pad_b), (0, 0)))
    bp = B + pad_b
    nb = bp // block_b

    out = pl.pallas_call(
        _anet_fused_kernel,
        out_shape=jax.ShapeDtypeStruct((bp, a_dim), jnp.float32),
        grid=(nb,),
        in_specs=[
            pl.BlockSpec((block_b, s_dim), lambda i: (i, 0)),
            pl.BlockSpec((s_dim, h_pad), lambda i: (0, 0)),
            pl.BlockSpec((1, h_pad), lambda i: (0, 0)),
            pl.BlockSpec((h_pad, a_dim), lambda i: (0, 0)),
            pl.BlockSpec((1, a_dim), lambda i: (0, 0)),
        ],
        out_specs=pl.BlockSpec((block_b, a_dim), lambda i: (i, 0)),
        compiler_params=pltpu.CompilerParams(
            dimension_semantics=("parallel",)),
    )(x, w1p, b1p, w2p, b2p)

    return out[:B]


# speedup vs baseline: 5.3892x; 1.1034x over previous
import jax
import jax.numpy as jnp
from jax.experimental import pallas as pl
from jax.experimental.pallas import tpu as pltpu

_HIDDEN = 30
_LANE = 128
_BLOCK_B = 8192


def _anet_fused_kernel(x_ref, w1_ref, b1_ref, w2_ref, b2_ref, o_ref):
    h = jnp.dot(x_ref[...], w1_ref[...], preferred_element_type=jnp.float32)
    h = jnp.maximum(h + b1_ref[...], 0.0)
    y = jnp.dot(h, w2_ref[...], preferred_element_type=jnp.float32)
    o_ref[...] = jnp.tanh(y + b2_ref[...]) * 2.0


def kernel(x, w1, b1, w2, b2):
    B, s_dim = x.shape
    a_dim = w2.shape[1]
    x = x.astype(jnp.float32)

    h_pad = max(_LANE, ((_HIDDEN + _LANE - 1) // _LANE) * _LANE)
    w1p = jnp.zeros((s_dim, h_pad), jnp.float32).at[:, :_HIDDEN].set(
        w1.astype(jnp.float32))
    b1p = jnp.zeros((1, h_pad), jnp.float32).at[:, :_HIDDEN].set(
        jnp.reshape(b1, (1, -1)).astype(jnp.float32))
    w2p = jnp.zeros((h_pad, a_dim), jnp.float32).at[:_HIDDEN, :].set(
        w2.astype(jnp.float32))
    b2p = jnp.reshape(b2, (1, a_dim)).astype(jnp.float32)

    block_b = min(_BLOCK_B, B)
    pad_b = (-B) % block_b
    if pad_b:
        x = jnp.pad(x, ((0, pad_b), (0, 0)))
    bp = B + pad_b
    nb = bp // block_b

    out = pl.pallas_call(
        _anet_fused_kernel,
        out_shape=jax.ShapeDtypeStruct((bp, a_dim), jnp.float32),
        grid=(nb,),
        in_specs=[
            pl.BlockSpec((block_b, s_dim), lambda i: (i, 0)),
            pl.BlockSpec((s_dim, h_pad), lambda i: (0, 0)),
            pl.BlockSpec((1, h_pad), lambda i: (0, 0)),
            pl.BlockSpec((h_pad, a_dim), lambda i: (0, 0)),
            pl.BlockSpec((1, a_dim), lambda i: (0, 0)),
        ],
        out_specs=pl.BlockSpec((block_b, a_dim), lambda i: (i, 0)),
        compiler_params=pltpu.CompilerParams(
            dimension_semantics=("parallel",)),
    )(x, w1p, b1p, w2p, b2p)

    return out[:B]


# speedup vs baseline: 5.4617x; 1.0135x over previous
import jax
import jax.numpy as jnp
from jax.experimental import pallas as pl
from jax.experimental.pallas import tpu as pltpu

_HIDDEN = 30
_LANE = 128
_BLOCK_B = 16384


def _anet_fused_kernel(x_ref, w1_ref, b1_ref, w2_ref, b2_ref, o_ref):
    h = jnp.dot(x_ref[...], w1_ref[...], preferred_element_type=jnp.float32)
    h = jnp.maximum(h + b1_ref[...], 0.0)
    y = jnp.dot(h, w2_ref[...], preferred_element_type=jnp.float32)
    o_ref[...] = jnp.tanh(y + b2_ref[...]) * 2.0


def kernel(x, w1, b1, w2, b2):
    B, s_dim = x.shape
    a_dim = w2.shape[1]
    x = x.astype(jnp.float32)

    h_pad = max(_LANE, ((_HIDDEN + _LANE - 1) // _LANE) * _LANE)
    w1p = jnp.zeros((s_dim, h_pad), jnp.float32).at[:, :_HIDDEN].set(
        w1.astype(jnp.float32))
    b1p = jnp.zeros((1, h_pad), jnp.float32).at[:, :_HIDDEN].set(
        jnp.reshape(b1, (1, -1)).astype(jnp.float32))
    w2p = jnp.zeros((h_pad, a_dim), jnp.float32).at[:_HIDDEN, :].set(
        w2.astype(jnp.float32))
    b2p = jnp.reshape(b2, (1, a_dim)).astype(jnp.float32)

    block_b = min(_BLOCK_B, B)
    pad_b = (-B) % block_b
    if pad_b:
        x = jnp.pad(x, ((0, pad_b), (0, 0)))
    bp = B + pad_b
    nb = bp // block_b

    out = pl.pallas_call(
        _anet_fused_kernel,
        out_shape=jax.ShapeDtypeStruct((bp, a_dim), jnp.float32),
        grid=(nb,),
        in_specs=[
            pl.BlockSpec((block_b, s_dim), lambda i: (i, 0)),
            pl.BlockSpec((s_dim, h_pad), lambda i: (0, 0)),
            pl.BlockSpec((1, h_pad), lambda i: (0, 0)),
            pl.BlockSpec((h_pad, a_dim), lambda i: (0, 0)),
            pl.BlockSpec((1, a_dim), lambda i: (0, 0)),
        ],
        out_specs=pl.BlockSpec((block_b, a_dim), lambda i: (i, 0)),
        compiler_params=pltpu.CompilerParams(
            dimension_semantics=("parallel",)),
    )(x, w1p, b1p, w2p, b2p)

    return out[:B]
